# baseline (speedup 1.0000x reference)
import jax
import jax.numpy as jnp
from jax.experimental import pallas as pl
from jax.experimental.pallas import tpu as pltpu

N = 512
D = 128
BLK = 128


def _grouping_kernel(x_ref, gt_ref, W_ref, b_ref, w_ref, bl_ref,
                     g_ref, bsum_ref, h_ref, d_scr):
    i = pl.program_id(0)

    @pl.when(i == 0)
    def _init():
        h_ref[...] = jnp.maximum(
            jnp.dot(x_ref[...], W_ref[...]) + b_ref[...], 0.0)
        bsum_ref[...] = jnp.zeros_like(bsum_ref)

    h = h_ref[...]
    hb = h_ref[pl.ds(i * BLK, BLK), :]
    diff = hb[:, None, :] - h[None, :, :]
    dm = (diff * diff).reshape(BLK * N, D)
    dv = jnp.dot(dm, w_ref[...])
    d_scr[...] = dv.reshape(BLK, N)
    d = d_scr[...] + bl_ref[0, 0]
    g = jax.nn.sigmoid(d)
    g_ref[...] = g

    gt = gt_ref[...]
    p = jnp.clip(g, 1e-7, 1.0 - 1e-7)
    bce = gt * jnp.log(p) + (1.0 - gt) * jnp.log(1.0 - p)
    bsum_ref[...] += jnp.sum(bce, keepdims=True)


def kernel(x, grouping_matrix_true, W_embed, b_embed, w_lin, b_lin):
    g, bsum = pl.pallas_call(
        _grouping_kernel,
        grid=(N // BLK,),
        in_specs=[
            pl.BlockSpec((N, D), lambda i: (0, 0)),
            pl.BlockSpec((BLK, N), lambda i: (i, 0)),
            pl.BlockSpec((D, D), lambda i: (0, 0)),
            pl.BlockSpec((1, D), lambda i: (0, 0)),
            pl.BlockSpec((D, 1), lambda i: (0, 0)),
            pl.BlockSpec((1, 1), lambda i: (0, 0)),
        ],
        out_specs=(
            pl.BlockSpec((BLK, N), lambda i: (i, 0)),
            pl.BlockSpec((1, 1), lambda i: (0, 0)),
        ),
        out_shape=(
            jax.ShapeDtypeStruct((N, N), jnp.float32),
            jax.ShapeDtypeStruct((1, 1), jnp.float32),
        ),
        scratch_shapes=[pltpu.VMEM((N, D), jnp.float32),
                        pltpu.VMEM((BLK, N), jnp.float32)],
    )(x, grouping_matrix_true, W_embed, b_embed.reshape(1, D),
      w_lin, b_lin.reshape(1, 1))

    grouping_loss = -bsum[0, 0] / (N * N)

    degree = jnp.sum(g, axis=1)
    lap = jnp.diag(degree) - g
    eigen_values, eigen_vectors = jnp.linalg.eigh(lap, symmetrize_input=False)
    aev = jnp.abs(eigen_values)
    i0 = jnp.argmin(aev)
    i1 = jnp.argmin(aev.at[i0].set(jnp.inf))
    fielder_value = eigen_values[i1]
    fielder_vector = eigen_vectors[i1]
    return grouping_loss, fielder_value, fielder_vector, g


# speedup vs baseline: 1.0020x; 1.0020x over previous
import jax
import jax.numpy as jnp
from jax.experimental import pallas as pl
from jax.experimental.pallas import tpu as pltpu

N = 512
D = 128
BLK = 128
T = N // BLK


def _grouping_kernel(x_ref, gt_ref, W_ref, b_ref, w_ref, bl_ref,
                     g_ref, bsum_ref, h_ref, d_scr):
    bi = pl.program_id(0)
    bj = pl.program_id(1)

    @pl.when((bi == 0) & (bj == 0))
    def _init():
        h_ref[...] = jnp.maximum(
            jnp.dot(x_ref[...], W_ref[...]) + b_ref[...], 0.0)
        bsum_ref[...] = jnp.zeros_like(bsum_ref)

    @pl.when(bj >= bi)
    def _compute():
        hi = h_ref[pl.ds(bi * BLK, BLK), :]
        hj = h_ref[pl.ds(bj * BLK, BLK), :]
        diff = hi[:, None, :] - hj[None, :, :]
        dm = (diff * diff).reshape(BLK * BLK, D)
        dv = jnp.dot(dm, w_ref[...])
        d_scr[...] = dv.reshape(BLK, BLK)
        d = d_scr[...] + bl_ref[0, 0]
        g = jax.nn.sigmoid(d)
        g_ref[pl.ds(bi * BLK, BLK), pl.ds(bj * BLK, BLK)] = g

        gt = gt_ref[pl.ds(bi * BLK, BLK), pl.ds(bj * BLK, BLK)]
        p = jnp.clip(g, 1e-7, 1.0 - 1e-7)
        bce = gt * jnp.log(p) + (1.0 - gt) * jnp.log(1.0 - p)
        bsum_ref[...] += jnp.sum(bce, keepdims=True)

        @pl.when(bj > bi)
        def _mirror():
            gT = g.T
            g_ref[pl.ds(bj * BLK, BLK), pl.ds(bi * BLK, BLK)] = gT
            gt2 = gt_ref[pl.ds(bj * BLK, BLK), pl.ds(bi * BLK, BLK)]
            p2 = jnp.clip(gT, 1e-7, 1.0 - 1e-7)
            bce2 = gt2 * jnp.log(p2) + (1.0 - gt2) * jnp.log(1.0 - p2)
            bsum_ref[...] += jnp.sum(bce2, keepdims=True)


def kernel(x, grouping_matrix_true, W_embed, b_embed, w_lin, b_lin):
    g, bsum = pl.pallas_call(
        _grouping_kernel,
        grid=(T, T),
        in_specs=[
            pl.BlockSpec((N, D), lambda i, j: (0, 0)),
            pl.BlockSpec((N, N), lambda i, j: (0, 0)),
            pl.BlockSpec((D, D), lambda i, j: (0, 0)),
            pl.BlockSpec((1, D), lambda i, j: (0, 0)),
            pl.BlockSpec((D, 1), lambda i, j: (0, 0)),
            pl.BlockSpec((1, 1), lambda i, j: (0, 0)),
        ],
        out_specs=(
            pl.BlockSpec((N, N), lambda i, j: (0, 0)),
            pl.BlockSpec((1, 1), lambda i, j: (0, 0)),
        ),
        out_shape=(
            jax.ShapeDtypeStruct((N, N), jnp.float32),
            jax.ShapeDtypeStruct((1, 1), jnp.float32),
        ),
        scratch_shapes=[pltpu.VMEM((N, D), jnp.float32),
                        pltpu.VMEM((BLK, BLK), jnp.float32)],
    )(x, grouping_matrix_true, W_embed, b_embed.reshape(1, D),
      w_lin, b_lin.reshape(1, 1))

    grouping_loss = -bsum[0, 0] / (N * N)

    degree = jnp.sum(g, axis=1)
    lap = jnp.diag(degree) - g
    eigen_values, eigen_vectors = jnp.linalg.eigh(lap, symmetrize_input=False)
    aev = jnp.abs(eigen_values)
    i0 = jnp.argmin(aev)
    i1 = jnp.argmin(aev.at[i0].set(jnp.inf))
    fielder_value = eigen_values[i1]
    fielder_vector = eigen_vectors[i1]
    return grouping_loss, fielder_value, fielder_vector, g
